# baseline (speedup 1.0000x reference)
import functools

import jax
import jax.numpy as jnp
from jax.experimental import pallas as pl
from jax.experimental.pallas import tpu as pltpu
from jax.experimental.pallas import tpu_sc as plsc

B = 16
D = 64
S = 32 * 32
K = 1024
N = B * S
COMMITMENT_COST = 0.25

BB = 2
M = BB * S


def _tc_a(z_ref, w_ref, idx_ref, loss_ref, acc_ref):
    b = pl.program_id(0)

    w = w_ref[...]

    zt = jnp.concatenate([z_ref[i].T for i in range(BB)], axis=0)

    mm2 = jax.lax.dot_general(
        zt, w + w, (((1,), (1,)), ((), ())),
        preferred_element_type=jnp.float32)
    zsq = jnp.sum(zt * zt, axis=1, keepdims=True)
    wsq = jnp.sum(w * w, axis=1).reshape(1, K)
    dist = (zsq + wsq) - mm2

    iota = jax.lax.broadcasted_iota(jnp.int32, (M, K), 1)
    mval = jnp.min(dist, axis=1, keepdims=True)
    idx = jnp.min(jnp.where(dist == mval, iota, K), axis=1).reshape(M, 1)
    idx_ref[0] = idx

    sq = jnp.sum(mval)

    @pl.when(b == 0)
    def _init():
        acc_ref[0, 0] = sq

    @pl.when(b > 0)
    def _acc():
        acc_ref[0, 0] += sq

    @pl.when(b == B // BB - 1)
    def _fin():
        loss = (1.0 + COMMITMENT_COST) * acc_ref[0, 0] / jnp.float32(N * D)
        loss_ref[...] = loss.reshape(1, 1)


_INFO = plsc.get_sparse_core_info()
_NC, _NS, _L = _INFO.num_cores, _INFO.num_subcores, _INFO.num_lanes
_NW = _NC * _NS
_BPW = N // _NW
_CHUNK = 128


def _sc_body(w_hbm, idx_hbm, qrows_hbm, hist_hbm, idx_v, rows_v, cnt_v, sem):
    wid = jax.lax.axis_index("s") * _NC + jax.lax.axis_index("c")
    base = wid * _BPW
    pltpu.sync_copy(idx_hbm.at[pl.ds(base, _BPW)], idx_v)
    copies = [
        pltpu.async_copy(
            w_hbm.at[idx_v.at[pl.ds(j * _CHUNK, _CHUNK)]],
            rows_v.at[pl.ds(j * _CHUNK, _CHUNK)],
            sem,
        )
        for j in range(_BPW // _CHUNK)
    ]
    for c in copies:
        c.wait()
    pltpu.sync_copy(rows_v, qrows_hbm.at[pl.ds(base, _BPW)])

    zeros = jnp.zeros((_L,), jnp.float32)
    for i in range(K // _L):
        cnt_v[pl.ds(i * _L, _L)] = zeros
    ones = jnp.ones((_L,), jnp.float32)
    for i in range(_BPW // _L):
        iv = idx_v[pl.ds(i * _L, _L)]
        plsc.addupdate_scatter(cnt_v, [iv], ones)
    pltpu.sync_copy(cnt_v, hist_hbm.at[wid])


_sc_gather = functools.partial(
    pl.kernel,
    mesh=plsc.VectorSubcoreMesh(core_axis_name="c", subcore_axis_name="s"),
    compiler_params=pltpu.CompilerParams(
        needs_layout_passes=False, use_tc_tiling_on_sc=False),
    out_type=[
        jax.ShapeDtypeStruct((N, D), jnp.float32),
        jax.ShapeDtypeStruct((_NW, K), jnp.float32),
    ],
    scratch_types=[
        pltpu.VMEM((_BPW,), jnp.int32),
        pltpu.VMEM((_BPW, D), jnp.float32),
        pltpu.VMEM((K,), jnp.float32),
        pltpu.SemaphoreType.DMA,
    ],
)(_sc_body)


def _tc_c(z_ref, qr_ref, hist_ref, out_ref, perp_ref):
    b = pl.program_id(0)
    for i in range(BB):
        z_i = z_ref[i]
        q = qr_ref[i].T
        out_ref[i] = z_i + (q - z_i)

    @pl.when(b == B // BB - 1)
    def _fin():
        counts = jnp.sum(hist_ref[...], axis=0, keepdims=True)
        probs = counts / jnp.float32(N)
        ent = -jnp.sum(probs * jnp.log(probs + 1e-10))
        perp_ref[...] = jnp.exp(ent).reshape(1, 1)


def kernel(z, W):
    z3 = z.reshape(B, D, S)
    idx_arr, loss = pl.pallas_call(
        _tc_a,
        grid=(B // BB,),
        in_specs=[
            pl.BlockSpec((BB, D, S), lambda b: (b, 0, 0)),
            pl.BlockSpec((K, D), lambda b: (0, 0)),
        ],
        out_specs=[
            pl.BlockSpec((1, M, 1), lambda b: (b, 0, 0)),
            pl.BlockSpec((1, 1), lambda b: (0, 0)),
        ],
        out_shape=[
            jax.ShapeDtypeStruct((B // BB, M, 1), jnp.int32),
            jax.ShapeDtypeStruct((1, 1), jnp.float32),
        ],
        scratch_shapes=[
            pltpu.SMEM((1, 1), jnp.float32),
        ],
    )(z3, W)

    qrows, hist = _sc_gather(W, idx_arr.reshape(N))

    out, perp = pl.pallas_call(
        _tc_c,
        grid=(B // BB,),
        in_specs=[
            pl.BlockSpec((BB, D, S), lambda b: (b, 0, 0)),
            pl.BlockSpec((BB, S, D), lambda b: (b, 0, 0)),
            pl.BlockSpec((_NW, K), lambda b: (0, 0)),
        ],
        out_specs=[
            pl.BlockSpec((BB, D, S), lambda b: (b, 0, 0)),
            pl.BlockSpec((1, 1), lambda b: (0, 0)),
        ],
        out_shape=[
            jax.ShapeDtypeStruct((B, D, S), jnp.float32),
            jax.ShapeDtypeStruct((1, 1), jnp.float32),
        ],
    )(z3, qrows.reshape(B, S, D), hist)

    return (out.reshape(B, D, 32, 32), loss[0, 0], perp[0, 0])


# speedup vs baseline: 1.6615x; 1.6615x over previous
import jax
import jax.numpy as jnp
from jax.experimental import pallas as pl
from jax.experimental.pallas import tpu as pltpu

B = 16
D = 64
S = 32 * 32
K = 1024
COMMITMENT_COST = 0.25


BB = 2
M = BB * S


def _vq_kernel(z_ref, w_ref, out_ref, loss_ref, perp_ref, counts_ref, acc_ref):
    b = pl.program_id(0)

    w = w_ref[...]

    zt = jnp.concatenate([z_ref[i].T for i in range(BB)], axis=0)

    mm2 = jax.lax.dot_general(
        zt, w + w, (((1,), (1,)), ((), ())),
        preferred_element_type=jnp.float32)
    zsq = jnp.sum(zt * zt, axis=1, keepdims=True)
    wsq = jnp.sum(w * w, axis=1).reshape(1, K)
    dist = (zsq + wsq) - mm2

    iota = jax.lax.broadcasted_iota(jnp.int32, (M, K), 1)
    mval = jnp.min(dist, axis=1, keepdims=True)
    idx = jnp.min(jnp.where(dist == mval, iota, K), axis=1).reshape(M, 1)

    onehot = (iota == idx).astype(jnp.float32)

    q = jax.lax.dot_general(
        w, onehot, (((0,), (1,)), ((), ())),
        preferred_element_type=jnp.float32)

    sq = jnp.float32(0.0)
    for i in range(BB):
        z_i = z_ref[i]
        diff = q[:, i * S:(i + 1) * S] - z_i
        out_ref[i] = z_i + diff
        sq += jnp.sum(diff * diff)

    counts_b = jax.lax.dot_general(
        jnp.ones((8, M), jnp.float32), onehot, (((1,), (0,)), ((), ())),
        preferred_element_type=jnp.float32)

    @pl.when(b == 0)
    def _init():
        acc_ref[0, 0] = sq
        counts_ref[...] = counts_b

    @pl.when(b > 0)
    def _acc():
        acc_ref[0, 0] += sq
        counts_ref[...] += counts_b

    @pl.when(b == B // BB - 1)
    def _fin():
        n = jnp.float32(B * S)
        loss = (1.0 + COMMITMENT_COST) * acc_ref[0, 0] / (n * D)
        loss_ref[...] = loss.reshape(1, 1)
        probs = counts_ref[0:1, :] / n
        ent = -jnp.sum(probs * jnp.log(probs + 1e-10))
        perp_ref[...] = jnp.exp(ent).reshape(1, 1)


def kernel(z, W):
    z3 = z.reshape(B, D, S)
    q, loss, perp = pl.pallas_call(
        _vq_kernel,
        grid=(B // BB,),
        in_specs=[
            pl.BlockSpec((BB, D, S), lambda b: (b, 0, 0)),
            pl.BlockSpec((K, D), lambda b: (0, 0)),
        ],
        out_specs=[
            pl.BlockSpec((BB, D, S), lambda b: (b, 0, 0)),
            pl.BlockSpec((1, 1), lambda b: (0, 0)),
            pl.BlockSpec((1, 1), lambda b: (0, 0)),
        ],
        out_shape=[
            jax.ShapeDtypeStruct((B, D, S), jnp.float32),
            jax.ShapeDtypeStruct((1, 1), jnp.float32),
            jax.ShapeDtypeStruct((1, 1), jnp.float32),
        ],
        scratch_shapes=[
            pltpu.VMEM((8, K), jnp.float32),
            pltpu.SMEM((1, 1), jnp.float32),
        ],
    )(z3, W)
    return (q.reshape(B, D, 32, 32), loss[0, 0], perp[0, 0])
